# Initial kernel scaffold; baseline (speedup 1.0000x reference)
import jax
import jax.numpy as jnp
from jax.experimental import pallas as pl


def kernel(node_features, edge_index_ids, edge_type_ids, node_type, node_score, emb_node_type, W_score, b_score, W_e1, b_e1, bn_gamma, bn_beta, bn_mean, bn_var, W_e2, b_e2, W_vh, b_vh, W_vx, b_vx):
    raise NotImplementedError("write your pallas kernel here")


# speedup vs baseline: 16.4114x; 16.4114x over previous
import functools

import jax
import jax.numpy as jnp
from jax import lax
from jax.experimental import pallas as pl
from jax.experimental.pallas import tpu as pltpu
from jax.experimental.pallas import tpu_sc as plsc

_NC = 2
_NS = 16
_NW = _NC * _NS
_CH = 128


def _gelu(v):
    return 0.5 * v * (1.0 + lax.erf(v * 0.7071067811865476))


def _matmul_t(a, w):
    return lax.dot_general(a, w, (((1,), (1,)), ((), ())),
                           preferred_element_type=jnp.float32)


def _feat_body(nt_ref, score_ref, emb_ref, ws_ref, bs_ref, js_ref, x_ref, y_ref):
    nt = nt_ref[:]
    rows = nt.shape[0]
    oh = (nt == lax.broadcasted_iota(jnp.int32, (1, 8), 1)).astype(jnp.float32)
    nte = _gelu(jnp.dot(oh, emb_ref[:], preferred_element_type=jnp.float32))
    enc = jnp.sin(score_ref[:] * js_ref[:])
    nse = _gelu(_matmul_t(enc, ws_ref[:]) + bs_ref[:])
    y_ref[:] = x_ref[:] + jnp.concatenate([nte, nse], axis=1)


def _table_body(we1_ref, be1_ref, g_ref, b_ref, m_ref, v_ref, we2_ref, be2_ref,
                t_ref, tself_ref):
    R = 1088
    r = lax.broadcasted_iota(jnp.int32, (R, 33), 0)
    c = lax.broadcasted_iota(jnp.int32, (R, 33), 1)
    et = r // 64
    ht = (r // 8) % 8
    tt = r % 8
    e_in = ((c == et) | (c == 17 + ht) | (c == 25 + tt)).astype(jnp.float32)
    h = _matmul_t(e_in, we1_ref[:]) + be1_ref[:]
    h = g_ref[:] * (h - m_ref[:]) * jax.lax.rsqrt(v_ref[:] + 1e-5) + b_ref[:]
    h = jnp.maximum(h, 0.0)
    t = _matmul_t(h, we2_ref[:]) + be2_ref[:]
    t_ref[:] = t
    sr = lax.broadcasted_iota(jnp.int32, (8, R), 0)
    sc = lax.broadcasted_iota(jnp.int32, (8, R), 1)
    sel = (sc == 1024 + 9 * sr).astype(jnp.float32)
    tself_ref[:] = jnp.dot(sel, t, preferred_element_type=jnp.float32)


def _out_body(x_ref, p0_ref, p1_ref, y_ref, nt_ref, tself_ref,
              wvh_ref, bvh_ref, wvx_ref, bvx_ref, o_ref):
    nt = nt_ref[:]
    oh = (nt == lax.broadcasted_iota(jnp.int32, (1, 8), 1)).astype(jnp.float32)
    agg = (p0_ref[:] + p1_ref[:] + y_ref[:]
           + jnp.dot(oh, tself_ref[:], preferred_element_type=jnp.float32))
    o_ref[:] = _gelu(_matmul_t(x_ref[:], wvh_ref[:]) + bvh_ref[:]
                     + _matmul_t(agg, wvx_ref[:]) + bvx_ref[:])


def _sc_body(ept, n_chunks, agg_rows,
             y_hbm, t_hbm, type_hbm, src_hbm, dst_hbm, et_hbm, zero_hbm,
             out_hbm,
             type_v, src_v, dst_v, et_v, combo_v, yrows, trows, agg_sh,
             sem1, sem2):
    c = lax.axis_index("c")
    s = lax.axis_index("s")
    rpt = agg_rows // _NS
    pltpu.sync_copy(zero_hbm.at[pl.ds(s * rpt, rpt)],
                    agg_sh.at[pl.ds(s * rpt, rpt)])
    pltpu.sync_copy(type_hbm, type_v)
    plsc.subcore_barrier()

    base = (c * _NS + s) * ept

    def chunk(k, carry):
        off = base + k * _CH
        pltpu.sync_copy(src_hbm.at[pl.ds(off, _CH)], src_v)
        pltpu.sync_copy(dst_hbm.at[pl.ds(off, _CH)], dst_v)
        pltpu.sync_copy(et_hbm.at[pl.ds(off, _CH)], et_v)
        for i in range(_CH // 16):
            sl = pl.ds(i * 16, 16)
            s16 = src_v[sl]
            d16 = dst_v[sl]
            ts = plsc.load_gather(type_v, [s16])
            td = plsc.load_gather(type_v, [d16])
            combo_v[sl] = et_v[sl] * 64 + ts * 8 + td
        cp1 = pltpu.async_copy(y_hbm.at[src_v], yrows, sem1)
        cp2 = pltpu.async_copy(t_hbm.at[combo_v], trows, sem2)
        cp1.wait()
        cp2.wait()
        pltpu.sync_copy(yrows, agg_sh.at[dst_v], add=True)
        pltpu.sync_copy(trows, agg_sh.at[dst_v], add=True)
        return carry

    lax.fori_loop(0, n_chunks, chunk, 0)
    plsc.subcore_barrier()
    pltpu.sync_copy(agg_sh.at[pl.ds(s * rpt, rpt)],
                    out_hbm.at[pl.ds(c * agg_rows + s * rpt, rpt)])


def kernel(node_features, edge_index_ids, edge_type_ids, node_type, node_score,
           emb_node_type, W_score, b_score, W_e1, b_e1, bn_gamma, bn_beta,
           bn_mean, bn_var, W_e2, b_e2, W_vh, b_vh, W_vx, b_vx):
    Bn, N, H = node_features.shape
    E = edge_index_ids.shape[1]
    NT = Bn * N

    x = node_features.reshape(NT, H)
    nt_col = node_type.reshape(NT, 1).astype(jnp.int32)
    score_col = node_score.reshape(NT, 1)
    js = jnp.power(1.1, jnp.arange(H // 2, dtype=jnp.float32)).reshape(1, H // 2)

    grid = 10
    rows = NT // grid
    row_spec = pl.BlockSpec((rows, H), lambda i: (i, 0))
    col_spec = pl.BlockSpec((rows, 1), lambda i: (i, 0))

    def full(a):
        return pl.BlockSpec(a.shape, lambda i: tuple(0 for _ in a.shape))

    y = pl.pallas_call(
        _feat_body,
        grid=(grid,),
        in_specs=[col_spec, col_spec, full(emb_node_type), full(W_score),
                  pl.BlockSpec((1, H // 2), lambda i: (0, 0)),
                  pl.BlockSpec((1, H // 2), lambda i: (0, 0)), row_spec],
        out_specs=row_spec,
        out_shape=jax.ShapeDtypeStruct((NT, H), jnp.float32),
    )(nt_col, score_col, emb_node_type, W_score, b_score.reshape(1, -1), js, x)

    b1 = b_e1.reshape(1, -1)
    T, T_self = pl.pallas_call(
        _table_body,
        out_shape=(jax.ShapeDtypeStruct((1088, H), jnp.float32),
                   jax.ShapeDtypeStruct((8, H), jnp.float32)),
    )(W_e1, b1, bn_gamma.reshape(1, -1), bn_beta.reshape(1, -1),
      bn_mean.reshape(1, -1), bn_var.reshape(1, -1), W_e2, b_e2.reshape(1, -1))

    ept = -(-E // (_NW * _CH)) * _CH
    n_chunks = ept // _CH
    e_pad = ept * _NW
    agg_rows = -(-(NT + 1) // (8 * _NS)) * (8 * _NS)

    src = edge_index_ids[0].astype(jnp.int32)
    dst = edge_index_ids[1].astype(jnp.int32)
    et = edge_type_ids.astype(jnp.int32)
    pad = e_pad - E
    pad_iota = jnp.arange(pad, dtype=jnp.int32)
    src_p = jnp.concatenate([src, pad_iota % NT])
    dst_p = jnp.concatenate([dst, NT + pad_iota % (agg_rows - NT)])
    et_p = jnp.concatenate([et, jnp.zeros((pad,), jnp.int32)])
    type_pad = jnp.concatenate(
        [node_type.reshape(NT).astype(jnp.int32),
         jnp.zeros((agg_rows - NT,), jnp.int32)])
    zeros_agg = jnp.zeros((agg_rows, H), jnp.float32)

    mesh = plsc.VectorSubcoreMesh(core_axis_name="c", subcore_axis_name="s",
                                  num_cores=_NC, num_subcores=_NS)
    partials = pl.kernel(
        functools.partial(_sc_body, ept, n_chunks, agg_rows),
        out_type=jax.ShapeDtypeStruct((_NC * agg_rows, H), jnp.float32),
        mesh=mesh,
        compiler_params=pltpu.CompilerParams(needs_layout_passes=False),
        scratch_types=[
            pltpu.VMEM((agg_rows,), jnp.int32),
            pltpu.VMEM((_CH,), jnp.int32),
            pltpu.VMEM((_CH,), jnp.int32),
            pltpu.VMEM((_CH,), jnp.int32),
            pltpu.VMEM((_CH,), jnp.int32),
            pltpu.VMEM((_CH, H), jnp.float32),
            pltpu.VMEM((_CH, H), jnp.float32),
            pltpu.VMEM_SHARED((agg_rows, H), jnp.float32),
            pltpu.SemaphoreType.DMA,
            pltpu.SemaphoreType.DMA,
        ],
    )(y, T, type_pad, src_p, dst_p, et_p, zeros_agg)

    p0 = partials[:NT]
    p1 = partials[agg_rows:agg_rows + NT]

    out = pl.pallas_call(
        _out_body,
        grid=(grid,),
        in_specs=[row_spec, row_spec, row_spec, row_spec, col_spec,
                  full(T_self), full(W_vh),
                  pl.BlockSpec((1, H), lambda i: (0, 0)), full(W_vx),
                  pl.BlockSpec((1, H), lambda i: (0, 0))],
        out_specs=row_spec,
        out_shape=jax.ShapeDtypeStruct((NT, H), jnp.float32),
    )(x, p0, p1, y, nt_col, T_self, W_vh, b_vh.reshape(1, -1), W_vx,
      b_vx.reshape(1, -1))

    return out.reshape(Bn, N, H)
